# Initial kernel scaffold; baseline (speedup 1.0000x reference)
import functools

import jax
import jax.numpy as jnp
from jax import lax
from jax.experimental import pallas as pl
from jax.experimental.pallas import tpu as pltpu
from jax.experimental.pallas import tpu_sc as plsc

N = 10000
E = 320000
D_IN = 128
D_HID = 128
D_OUT = 64
K = 10
ALPHA = 0.1

NPAD = 10240
TPN = NPAD // 16
FH = 32
DUMMY = N

EPT = 20480
EPAD = 16 * EPT
CH = 1024
NCH = EPT // CH

EPT2 = EPAD // 32
NCH2 = EPT2 // CH

_f32 = jnp.float32
_i32 = jnp.int32


def _mlp_body(x_ref, w1_ref, b1_ref, w2_ref, b2_ref, h_ref):
    hmid = jnp.dot(x_ref[...], w1_ref[...],
                   preferred_element_type=jnp.float32,
                   precision=lax.Precision.HIGHEST)
    hmid = jnp.maximum(hmid + b1_ref[...], 0.0)
    h = jnp.dot(hmid, w2_ref[...],
                preferred_element_type=jnp.float32,
                precision=lax.Precision.HIGHEST)
    h_ref[...] = h + b2_ref[...]


def _mlp(xp, W1, b1, W2, b2):
    return pl.pallas_call(
        _mlp_body,
        out_shape=jax.ShapeDtypeStruct((NPAD, D_OUT), _f32),
    )(xp, W1, b1.reshape(1, D_HID), W2, b2.reshape(1, D_OUT))


def _prep_body(h_ref, degp_ref, p0_ref, a_ref, sq_ref):
    deg = degp_ref[0, :] + degp_ref[1, :] + 1.0
    dinv = lax.rsqrt(deg)
    a_ref[0, :] = (1.0 - ALPHA) * dinv * dinv
    sq_ref[0, :] = deg * dinv
    p0 = h_ref[...] * dinv[:, None]
    p0_ref[0] = p0[:, :FH]
    p0_ref[1] = p0[:, FH:]


def _prep(h, degp):
    return pl.pallas_call(
        _prep_body,
        out_shape=[
            jax.ShapeDtypeStruct((2, NPAD, FH), _f32),
            jax.ShapeDtypeStruct((1, NPAD), _f32),
            jax.ShapeDtypeStruct((1, NPAD), _f32),
        ],
    )(h, degp)


def _deg_kernel(col2):
    mesh = plsc.VectorSubcoreMesh(core_axis_name="c", subcore_axis_name="s")

    @functools.partial(
        pl.kernel,
        out_type=jax.ShapeDtypeStruct((2 * NPAD,), _f32),
        mesh=mesh,
        scratch_types=[
            pltpu.VMEM_SHARED((NPAD,), _f32),
            pltpu.VMEM((CH,), _f32),
            pltpu.VMEM((TPN,), _f32),
            pltpu.VMEM((8, 128), _i32),
        ],
    )
    def k(col_hbm, deg_hbm, cnt_sh, ones_v, stage_v, idx_v):
        cid = lax.axis_index("c")
        sid = lax.axis_index("s")
        base = sid * TPN

        @pl.loop(0, CH, step=16)
        def _(i):
            ones_v[pl.ds(i, 16)] = jnp.ones((16,), _f32)

        @pl.loop(0, TPN, step=16)
        def _(i):
            stage_v[pl.ds(i, 16)] = jnp.zeros((16,), _f32)

        pltpu.sync_copy(stage_v, cnt_sh.at[pl.ds(base, TPN)])
        plsc.subcore_barrier()

        @pl.loop(0, NCH2)
        def _(c):
            pltpu.sync_copy(col_hbm.at[cid, sid, c], idx_v)
            pltpu.sync_copy(ones_v, cnt_sh.at[idx_v], add=True)

        plsc.subcore_barrier()
        pltpu.sync_copy(cnt_sh.at[pl.ds(base, TPN)], stage_v)
        pltpu.sync_copy(stage_v, deg_hbm.at[pl.ds(cid * NPAD + base, TPN)])

    return k(col2)


def _prop_kernel(p0s, a, sq, row4, col4):
    mesh = plsc.VectorSubcoreMesh(core_axis_name="c", subcore_axis_name="s")

    @functools.partial(
        pl.kernel,
        out_type=[
            jax.ShapeDtypeStruct((2 * NPAD, FH), _f32),
            jax.ShapeDtypeStruct((2 * NPAD, FH), _f32),
        ],
        mesh=mesh,
        scratch_types=[
            pltpu.VMEM_SHARED((NPAD, FH), _f32),
            pltpu.VMEM((TPN, FH), _f32),
            pltpu.VMEM((TPN, FH), _f32),
            pltpu.VMEM((TPN, FH), _f32),
            pltpu.VMEM((TPN,), _f32),
            pltpu.VMEM((CH, FH), _f32),
            pltpu.VMEM((8, 128), _i32),
            pltpu.VMEM((8, 128), _i32),
            pltpu.SemaphoreType.DMA,
        ],
    )
    def k(p0_hbm, a_hbm, sq_hbm, row_hbm, col_hbm, out_hbm, p_hbm,
          t_sh, p_v, p0_v, t_v, a_v, g_v, ri_v, ci_v, sem):
        cid = lax.axis_index("c")
        sid = lax.axis_index("s")
        base = sid * TPN
        gbase = cid * NPAD + base

        pltpu.sync_copy(p0_hbm.at[pl.ds(gbase, TPN)], p0_v)
        pltpu.sync_copy(a_hbm.at[pl.ds(base, TPN)], a_v)

        pltpu.sync_copy(p0_v, p_hbm.at[pl.ds(gbase, TPN)])
        pltpu.sync_copy(p0_v, t_sh.at[pl.ds(base, TPN)])
        pltpu.sync_copy(p0_v, p_v)
        plsc.subcore_barrier()

        @pl.loop(0, K)
        def _(step):
            @pl.loop(0, NCH)
            def _(c):
                pltpu.sync_copy(row_hbm.at[cid, sid, c], ri_v)
                pltpu.sync_copy(col_hbm.at[sid, c], ci_v)
                pltpu.async_copy(p_hbm.at[ri_v], g_v, sem).wait()
                pltpu.sync_copy(g_v, t_sh.at[ci_v], add=True)

            plsc.subcore_barrier()

            pltpu.sync_copy(t_sh.at[pl.ds(base, TPN)], t_v)

            @pl.loop(0, TPN)
            def _(r):
                a_s = a_v[r]
                p_v[r, pl.ds(0, 16)] = (t_v[r, pl.ds(0, 16)] * a_s
                                        + p0_v[r, pl.ds(0, 16)] * ALPHA)
                p_v[r, pl.ds(16, 16)] = (t_v[r, pl.ds(16, 16)] * a_s
                                         + p0_v[r, pl.ds(16, 16)] * ALPHA)

            pltpu.sync_copy(p_v, p_hbm.at[pl.ds(gbase, TPN)])
            pltpu.sync_copy(p_v, t_sh.at[pl.ds(base, TPN)])
            plsc.subcore_barrier()

        pltpu.sync_copy(sq_hbm.at[pl.ds(base, TPN)], a_v)

        @pl.loop(0, TPN)
        def _(r):
            s_s = a_v[r]
            p_v[r, pl.ds(0, 16)] = p_v[r, pl.ds(0, 16)] * s_s
            p_v[r, pl.ds(16, 16)] = p_v[r, pl.ds(16, 16)] * s_s

        pltpu.sync_copy(p_v, out_hbm.at[pl.ds(gbase, TPN)])

    return k(p0s, a, sq, row4, col4)


def kernel(x, edge_index, W1, b1, W2, b2):
    row = edge_index[0].astype(_i32)
    col = edge_index[1].astype(_i32)
    pad = jnp.full((EPAD - E,), DUMMY, _i32)
    rowp = jnp.concatenate([row, pad])
    colp = jnp.concatenate([col, pad])

    col2 = colp.reshape(2, 16, NCH2, 8, 128)
    col4 = colp.reshape(16, NCH, 8, 128)
    shift = (jnp.arange(2, dtype=_i32) * NPAD).reshape(2, 1, 1, 1, 1)
    row4 = rowp.reshape(1, 16, NCH, 8, 128) + shift

    xp = jnp.pad(x, ((0, NPAD - N), (0, 0)))

    h = _mlp(xp, W1, b1, W2, b2)
    degp = _deg_kernel(col2)
    p0s, a, sq = _prep(h, degp.reshape(2, NPAD))
    out2, _ = _prop_kernel(
        p0s.reshape(2 * NPAD, FH), a.reshape(NPAD), sq.reshape(NPAD),
        row4, col4)

    out2 = out2.reshape(2, NPAD, FH)
    return jnp.concatenate([out2[0, :N, :], out2[1, :N, :]], axis=1)


# speedup vs baseline: 13.7095x; 13.7095x over previous
import functools

import jax
import jax.numpy as jnp
from jax import lax
from jax.experimental import pallas as pl
from jax.experimental.pallas import tpu as pltpu
from jax.experimental.pallas import tpu_sc as plsc

N = 10000
E = 320000
D_IN = 128
D_HID = 128
D_OUT = 64
K = 10
ALPHA = 0.1

NPAD = 10240
TPN = NPAD // 16
FH = 32
DUMMY = N

EPT = 20480
EPAD = 16 * EPT
CH = 1024
NCH = EPT // CH

EPT2 = EPAD // 32
NCH2 = EPT2 // CH

_f32 = jnp.float32
_i32 = jnp.int32


def _mlp_body(x_ref, w1_ref, b1_ref, w2_ref, b2_ref, h_ref):
    hmid = jnp.dot(x_ref[...], w1_ref[...],
                   preferred_element_type=jnp.float32,
                   precision=lax.Precision.HIGHEST)
    hmid = jnp.maximum(hmid + b1_ref[...], 0.0)
    h = jnp.dot(hmid, w2_ref[...],
                preferred_element_type=jnp.float32,
                precision=lax.Precision.HIGHEST)
    h_ref[...] = h + b2_ref[...]


def _mlp(xp, W1, b1, W2, b2):
    return pl.pallas_call(
        _mlp_body,
        out_shape=jax.ShapeDtypeStruct((NPAD, D_OUT), _f32),
    )(xp, W1, b1.reshape(1, D_HID), W2, b2.reshape(1, D_OUT))


def _prep_body(h_ref, degp_ref, p0_ref, a2_ref, sq2_ref):
    deg = degp_ref[0, :] + degp_ref[1, :] + 1.0
    dinv = lax.rsqrt(deg)
    a2_ref[...] = jnp.broadcast_to(
        ((1.0 - ALPHA) * dinv * dinv)[:, None], (NPAD, FH))
    sq2_ref[...] = jnp.broadcast_to((deg * dinv)[:, None], (NPAD, FH))
    p0 = h_ref[...] * dinv[:, None]
    p0_ref[0] = p0[:, :FH]
    p0_ref[1] = p0[:, FH:]


def _prep(h, degp):
    return pl.pallas_call(
        _prep_body,
        out_shape=[
            jax.ShapeDtypeStruct((2, NPAD, FH), _f32),
            jax.ShapeDtypeStruct((NPAD, FH), _f32),
            jax.ShapeDtypeStruct((NPAD, FH), _f32),
        ],
    )(h, degp)


def _deg_kernel(col2):
    mesh = plsc.VectorSubcoreMesh(core_axis_name="c", subcore_axis_name="s")

    @functools.partial(
        pl.kernel,
        out_type=jax.ShapeDtypeStruct((2 * NPAD,), _f32),
        mesh=mesh,
        compiler_params=pltpu.CompilerParams(use_tc_tiling_on_sc=False),
        scratch_types=[
            pltpu.VMEM_SHARED((NPAD,), _f32),
            pltpu.VMEM((CH,), _f32),
            pltpu.VMEM((TPN,), _f32),
            pltpu.VMEM((CH,), _i32),
        ],
    )
    def k(col_hbm, deg_hbm, cnt_sh, ones_v, stage_v, idx_v):
        cid = lax.axis_index("c")
        sid = lax.axis_index("s")
        base = sid * TPN

        @pl.loop(0, CH, step=16)
        def _(i):
            ones_v[pl.ds(i, 16)] = jnp.ones((16,), _f32)

        @pl.loop(0, TPN, step=16)
        def _(i):
            stage_v[pl.ds(i, 16)] = jnp.zeros((16,), _f32)

        pltpu.sync_copy(stage_v, cnt_sh.at[pl.ds(base, TPN)])
        plsc.subcore_barrier()

        @pl.loop(0, NCH2)
        def _(c):
            pltpu.sync_copy(col_hbm.at[cid, sid, c], idx_v)
            pltpu.sync_copy(ones_v, cnt_sh.at[idx_v], add=True)

        plsc.subcore_barrier()
        pltpu.sync_copy(cnt_sh.at[pl.ds(base, TPN)], stage_v)
        pltpu.sync_copy(stage_v, deg_hbm.at[pl.ds(cid * NPAD + base, TPN)])

    return k(col2)


def _prop_kernel(p0s, a2, sq2, row4, col4):
    mesh = plsc.VectorSubcoreMesh(core_axis_name="c", subcore_axis_name="s")

    @functools.partial(
        pl.kernel,
        out_type=[
            jax.ShapeDtypeStruct((2 * NPAD, FH), _f32),
            jax.ShapeDtypeStruct((2 * NPAD, FH), _f32),
        ],
        mesh=mesh,
        compiler_params=pltpu.CompilerParams(use_tc_tiling_on_sc=False),
        scratch_types=[
            pltpu.VMEM_SHARED((NPAD, FH), _f32),
            pltpu.VMEM((TPN, FH), _f32),
            pltpu.VMEM((TPN, FH), _f32),
            pltpu.VMEM((TPN, FH), _f32),
            pltpu.VMEM((CH, FH), _f32),
            pltpu.VMEM((CH,), _i32),
            pltpu.VMEM((CH,), _i32),
            pltpu.SemaphoreType.DMA,
        ],
    )
    def k(p0_hbm, a2_hbm, sq2_hbm, row_hbm, col_hbm, out_hbm, p_hbm,
          t_sh, p_v, p0_v, a2_v, g_v, ri_v, ci_v, sem):
        cid = lax.axis_index("c")
        sid = lax.axis_index("s")
        base = sid * TPN
        gbase = cid * NPAD + base

        pltpu.sync_copy(p0_hbm.at[pl.ds(gbase, TPN)], p0_v)
        pltpu.sync_copy(a2_hbm.at[pl.ds(base, TPN)], a2_v)

        pltpu.sync_copy(p0_v, p_hbm.at[pl.ds(gbase, TPN)])
        pltpu.sync_copy(p0_v, t_sh.at[pl.ds(base, TPN)])
        plsc.subcore_barrier()

        @pl.loop(0, K)
        def _(step):
            @pl.loop(0, NCH)
            def _(c):
                pltpu.sync_copy(row_hbm.at[cid, sid, c], ri_v)
                pltpu.sync_copy(col_hbm.at[sid, c], ci_v)
                pltpu.async_copy(p_hbm.at[ri_v], g_v, sem).wait()
                pltpu.sync_copy(g_v, t_sh.at[ci_v], add=True)

            plsc.subcore_barrier()

            pltpu.sync_copy(t_sh.at[pl.ds(base, TPN)], g_v.at[pl.ds(0, TPN)])

            @pl.loop(0, TPN)
            def _(r):
                p_v[r, pl.ds(0, 16)] = (g_v[r, pl.ds(0, 16)]
                                        * a2_v[r, pl.ds(0, 16)]
                                        + p0_v[r, pl.ds(0, 16)] * ALPHA)
                p_v[r, pl.ds(16, 16)] = (g_v[r, pl.ds(16, 16)]
                                         * a2_v[r, pl.ds(16, 16)]
                                         + p0_v[r, pl.ds(16, 16)] * ALPHA)

            pltpu.sync_copy(p_v, p_hbm.at[pl.ds(gbase, TPN)])
            pltpu.sync_copy(p_v, t_sh.at[pl.ds(base, TPN)])
            plsc.subcore_barrier()

        pltpu.sync_copy(sq2_hbm.at[pl.ds(base, TPN)], g_v.at[pl.ds(0, TPN)])

        @pl.loop(0, TPN)
        def _(r):
            p_v[r, pl.ds(0, 16)] = p_v[r, pl.ds(0, 16)] * g_v[r, pl.ds(0, 16)]
            p_v[r, pl.ds(16, 16)] = (p_v[r, pl.ds(16, 16)]
                                     * g_v[r, pl.ds(16, 16)])

        pltpu.sync_copy(p_v, out_hbm.at[pl.ds(gbase, TPN)])

    return k(p0s, a2, sq2, row4, col4)


def kernel(x, edge_index, W1, b1, W2, b2):
    row = edge_index[0].astype(_i32)
    col = edge_index[1].astype(_i32)
    pad = jnp.full((EPAD - E,), DUMMY, _i32)
    rowp = jnp.concatenate([row, pad])
    colp = jnp.concatenate([col, pad])

    col2 = colp.reshape(2, 16, NCH2, CH)
    col4 = colp.reshape(16, NCH, CH)
    shift = (jnp.arange(2, dtype=_i32) * NPAD).reshape(2, 1, 1, 1)
    row4 = rowp.reshape(1, 16, NCH, CH) + shift

    xp = jnp.pad(x, ((0, NPAD - N), (0, 0)))

    h = _mlp(xp, W1, b1, W2, b2)
    degp = _deg_kernel(col2)
    p0s, a2, sq2 = _prep(h, degp.reshape(2, NPAD))
    out2, _ = _prop_kernel(
        p0s.reshape(2 * NPAD, FH), a2, sq2, row4, col4)

    out2 = out2.reshape(2, NPAD, FH)
    return jnp.concatenate([out2[0, :N, :], out2[1, :N, :]], axis=1)


# speedup vs baseline: 16.0776x; 1.1727x over previous
import functools

import jax
import jax.numpy as jnp
from jax import lax
from jax.experimental import pallas as pl
from jax.experimental.pallas import tpu as pltpu
from jax.experimental.pallas import tpu_sc as plsc

N = 10000
E = 320000
D_IN = 128
D_HID = 128
D_OUT = 64
K = 10
ALPHA = 0.1

NPAD = 10240
TPN = NPAD // 16
FH = 32
DUMMY = N

EPT = 20480
EPAD = 16 * EPT
CH = 1024
CHB = 512
NCHB = EPT // CHB

EPT2 = EPAD // 32
NCH2 = EPT2 // CH

_f32 = jnp.float32
_i32 = jnp.int32


def _mlp_body(x_ref, w1_ref, b1_ref, w2_ref, b2_ref, h_ref):
    hmid = jnp.dot(x_ref[...], w1_ref[...],
                   preferred_element_type=jnp.float32,
                   precision=lax.Precision.HIGHEST)
    hmid = jnp.maximum(hmid + b1_ref[...], 0.0)
    h = jnp.dot(hmid, w2_ref[...],
                preferred_element_type=jnp.float32,
                precision=lax.Precision.HIGHEST)
    h_ref[...] = h + b2_ref[...]


def _mlp(xp, W1, b1, W2, b2):
    return pl.pallas_call(
        _mlp_body,
        out_shape=jax.ShapeDtypeStruct((NPAD, D_OUT), _f32),
    )(xp, W1, b1.reshape(1, D_HID), W2, b2.reshape(1, D_OUT))


def _prep_body(h_ref, degp_ref, p0_ref, a2_ref, sq2_ref):
    deg = degp_ref[0, :] + degp_ref[1, :] + 1.0
    dinv = lax.rsqrt(deg)
    a2_ref[...] = jnp.broadcast_to(
        ((1.0 - ALPHA) * dinv * dinv)[:, None], (NPAD, FH))
    sq2_ref[...] = jnp.broadcast_to((deg * dinv)[:, None], (NPAD, FH))
    p0 = h_ref[...] * dinv[:, None]
    p0_ref[0] = p0[:, :FH]
    p0_ref[1] = p0[:, FH:]


def _prep(h, degp):
    return pl.pallas_call(
        _prep_body,
        out_shape=[
            jax.ShapeDtypeStruct((2, NPAD, FH), _f32),
            jax.ShapeDtypeStruct((NPAD, FH), _f32),
            jax.ShapeDtypeStruct((NPAD, FH), _f32),
        ],
    )(h, degp)


def _deg_kernel(col2):
    mesh = plsc.VectorSubcoreMesh(core_axis_name="c", subcore_axis_name="s")

    @functools.partial(
        pl.kernel,
        out_type=jax.ShapeDtypeStruct((2 * NPAD,), _f32),
        mesh=mesh,
        compiler_params=pltpu.CompilerParams(use_tc_tiling_on_sc=False),
        scratch_types=[
            pltpu.VMEM_SHARED((NPAD,), _f32),
            pltpu.VMEM((CH,), _f32),
            pltpu.VMEM((TPN,), _f32),
            pltpu.VMEM((CH,), _i32),
        ],
    )
    def k(col_hbm, deg_hbm, cnt_sh, ones_v, stage_v, idx_v):
        cid = lax.axis_index("c")
        sid = lax.axis_index("s")
        base = sid * TPN

        @pl.loop(0, CH, step=16)
        def _(i):
            ones_v[pl.ds(i, 16)] = jnp.ones((16,), _f32)

        @pl.loop(0, TPN, step=16)
        def _(i):
            stage_v[pl.ds(i, 16)] = jnp.zeros((16,), _f32)

        pltpu.sync_copy(stage_v, cnt_sh.at[pl.ds(base, TPN)])
        plsc.subcore_barrier()

        @pl.loop(0, NCH2)
        def _(c):
            pltpu.sync_copy(col_hbm.at[cid, sid, c], idx_v)
            pltpu.sync_copy(ones_v, cnt_sh.at[idx_v], add=True)

        plsc.subcore_barrier()
        pltpu.sync_copy(cnt_sh.at[pl.ds(base, TPN)], stage_v)
        pltpu.sync_copy(stage_v, deg_hbm.at[pl.ds(cid * NPAD + base, TPN)])

    return k(col2)


def _prop_kernel(p0s, a2, sq2, row4, col4):
    mesh = plsc.VectorSubcoreMesh(core_axis_name="c", subcore_axis_name="s")

    @functools.partial(
        pl.kernel,
        out_type=[
            jax.ShapeDtypeStruct((2 * NPAD, FH), _f32),
            jax.ShapeDtypeStruct((2 * NPAD, FH), _f32),
        ],
        mesh=mesh,
        compiler_params=pltpu.CompilerParams(use_tc_tiling_on_sc=False),
        scratch_types=[
            pltpu.VMEM_SHARED((NPAD, FH), _f32),
            pltpu.VMEM((TPN, FH), _f32),
            pltpu.VMEM((TPN, FH), _f32),
            pltpu.VMEM((TPN, FH), _f32),
            pltpu.VMEM((CHB, FH), _f32),
            pltpu.VMEM((CHB, FH), _f32),
            pltpu.VMEM((CHB,), _i32),
            pltpu.VMEM((CHB,), _i32),
            pltpu.VMEM((CHB,), _i32),
            pltpu.VMEM((CHB,), _i32),
            pltpu.SemaphoreType.DMA,
            pltpu.SemaphoreType.DMA,
        ],
    )
    def k(p0_hbm, a2_hbm, sq2_hbm, row_hbm, col_hbm, out_hbm, p_hbm,
          t_sh, p_v, p0_v, a2_v, gA, gB, riA, ciA, riB, ciB, semA, semB):
        cid = lax.axis_index("c")
        sid = lax.axis_index("s")
        base = sid * TPN
        gbase = cid * NPAD + base

        pltpu.sync_copy(p0_hbm.at[pl.ds(gbase, TPN)], p0_v)
        pltpu.sync_copy(a2_hbm.at[pl.ds(base, TPN)], a2_v)

        pltpu.sync_copy(p0_v, p_hbm.at[pl.ds(gbase, TPN)])
        pltpu.sync_copy(p0_v, t_sh.at[pl.ds(base, TPN)])
        plsc.subcore_barrier()

        @pl.loop(0, K)
        def _(step):
            pltpu.sync_copy(row_hbm.at[cid, sid, 0], riA)
            pltpu.sync_copy(col_hbm.at[sid, 0], ciA)
            pltpu.async_copy(p_hbm.at[riA], gA, semA)

            @pl.loop(0, NCHB, step=2)
            def _(c):
                pltpu.sync_copy(row_hbm.at[cid, sid, c + 1], riB)
                pltpu.sync_copy(col_hbm.at[sid, c + 1], ciB)
                pltpu.async_copy(p_hbm.at[riB], gB, semB)
                pltpu.make_async_copy(p_hbm.at[riA], gA, semA).wait()
                pltpu.sync_copy(gA, t_sh.at[ciA], add=True)

                @pl.when(c + 2 < NCHB)
                def _():
                    pltpu.sync_copy(row_hbm.at[cid, sid, c + 2], riA)
                    pltpu.sync_copy(col_hbm.at[sid, c + 2], ciA)
                    pltpu.async_copy(p_hbm.at[riA], gA, semA)

                pltpu.make_async_copy(p_hbm.at[riB], gB, semB).wait()
                pltpu.sync_copy(gB, t_sh.at[ciB], add=True)

            plsc.subcore_barrier()

            pltpu.sync_copy(t_sh.at[pl.ds(base, CHB)], gA)
            pltpu.sync_copy(t_sh.at[pl.ds(base + CHB, TPN - CHB)],
                            gB.at[pl.ds(0, TPN - CHB)])

            @pl.loop(0, CHB)
            def _(r):
                p_v[r, pl.ds(0, 16)] = (gA[r, pl.ds(0, 16)]
                                        * a2_v[r, pl.ds(0, 16)]
                                        + p0_v[r, pl.ds(0, 16)] * ALPHA)
                p_v[r, pl.ds(16, 16)] = (gA[r, pl.ds(16, 16)]
                                         * a2_v[r, pl.ds(16, 16)]
                                         + p0_v[r, pl.ds(16, 16)] * ALPHA)

            @pl.loop(CHB, TPN)
            def _(r):
                p_v[r, pl.ds(0, 16)] = (gB[r - CHB, pl.ds(0, 16)]
                                        * a2_v[r, pl.ds(0, 16)]
                                        + p0_v[r, pl.ds(0, 16)] * ALPHA)
                p_v[r, pl.ds(16, 16)] = (gB[r - CHB, pl.ds(16, 16)]
                                         * a2_v[r, pl.ds(16, 16)]
                                         + p0_v[r, pl.ds(16, 16)] * ALPHA)

            pltpu.sync_copy(p_v, p_hbm.at[pl.ds(gbase, TPN)])
            pltpu.sync_copy(p_v, t_sh.at[pl.ds(base, TPN)])
            plsc.subcore_barrier()

        pltpu.sync_copy(sq2_hbm.at[pl.ds(base, CHB)], gA)
        pltpu.sync_copy(sq2_hbm.at[pl.ds(base + CHB, TPN - CHB)],
                        gB.at[pl.ds(0, TPN - CHB)])

        @pl.loop(0, CHB)
        def _(r):
            p_v[r, pl.ds(0, 16)] = p_v[r, pl.ds(0, 16)] * gA[r, pl.ds(0, 16)]
            p_v[r, pl.ds(16, 16)] = (p_v[r, pl.ds(16, 16)]
                                     * gA[r, pl.ds(16, 16)])

        @pl.loop(CHB, TPN)
        def _(r):
            p_v[r, pl.ds(0, 16)] = (p_v[r, pl.ds(0, 16)]
                                    * gB[r - CHB, pl.ds(0, 16)])
            p_v[r, pl.ds(16, 16)] = (p_v[r, pl.ds(16, 16)]
                                     * gB[r - CHB, pl.ds(16, 16)])

        pltpu.sync_copy(p_v, out_hbm.at[pl.ds(gbase, TPN)])

    return k(p0s, a2, sq2, row4, col4)


def kernel(x, edge_index, W1, b1, W2, b2):
    row = edge_index[0].astype(_i32)
    col = edge_index[1].astype(_i32)
    pad = jnp.full((EPAD - E,), DUMMY, _i32)
    rowp = jnp.concatenate([row, pad])
    colp = jnp.concatenate([col, pad])

    col2 = colp.reshape(2, 16, NCH2, CH)
    col4 = colp.reshape(16, NCHB, CHB)
    shift = (jnp.arange(2, dtype=_i32) * NPAD).reshape(2, 1, 1, 1)
    row4 = rowp.reshape(1, 16, NCHB, CHB) + shift

    xp = jnp.pad(x, ((0, NPAD - N), (0, 0)))

    h = _mlp(xp, W1, b1, W2, b2)
    degp = _deg_kernel(col2)
    p0s, a2, sq2 = _prep(h, degp.reshape(2, NPAD))
    out2, _ = _prop_kernel(
        p0s.reshape(2 * NPAD, FH), a2, sq2, row4, col4)

    out2 = out2.reshape(2, NPAD, FH)
    return jnp.concatenate([out2[0, :N, :], out2[1, :N, :]], axis=1)


# speedup vs baseline: 20.0420x; 1.2466x over previous
import functools

import jax
import jax.numpy as jnp
from jax import lax
from jax.experimental import pallas as pl
from jax.experimental.pallas import tpu as pltpu
from jax.experimental.pallas import tpu_sc as plsc

N = 10000
E = 320000
D_IN = 128
D_HID = 128
D_OUT = 64
K = 10
ALPHA = 0.1

NPAD = 10240
TPN = NPAD // 16
FH = 32
DUMMY = N

EPT = 20480
EPAD = 16 * EPT
CH = 1024
CHB = 320
NCHB = EPT // CHB

EPT2 = EPAD // 32
NCH2 = EPT2 // CH

_f32 = jnp.float32
_i32 = jnp.int32


def _mlp_body(x_ref, w1_ref, b1_ref, w2_ref, b2_ref, h_ref):
    hmid = jnp.dot(x_ref[...], w1_ref[...],
                   preferred_element_type=jnp.float32,
                   precision=lax.Precision.HIGHEST)
    hmid = jnp.maximum(hmid + b1_ref[...], 0.0)
    h = jnp.dot(hmid, w2_ref[...],
                preferred_element_type=jnp.float32,
                precision=lax.Precision.HIGHEST)
    h_ref[...] = h + b2_ref[...]


def _mlp(xp, W1, b1, W2, b2):
    return pl.pallas_call(
        _mlp_body,
        out_shape=jax.ShapeDtypeStruct((NPAD, D_OUT), _f32),
    )(xp, W1, b1.reshape(1, D_HID), W2, b2.reshape(1, D_OUT))


def _prep_body(h_ref, degp_ref, p0_ref, a2_ref, sq2_ref):
    deg = degp_ref[0, :] + degp_ref[1, :] + 1.0
    dinv = lax.rsqrt(deg)
    a2_ref[...] = jnp.broadcast_to(
        ((1.0 - ALPHA) * dinv * dinv)[:, None], (NPAD, FH))
    sq2_ref[...] = jnp.broadcast_to((deg * dinv)[:, None], (NPAD, FH))
    p0 = h_ref[...] * dinv[:, None]
    p0_ref[0] = p0[:, :FH]
    p0_ref[1] = p0[:, FH:]


def _prep(h, degp):
    return pl.pallas_call(
        _prep_body,
        out_shape=[
            jax.ShapeDtypeStruct((2, NPAD, FH), _f32),
            jax.ShapeDtypeStruct((NPAD, FH), _f32),
            jax.ShapeDtypeStruct((NPAD, FH), _f32),
        ],
    )(h, degp)


def _deg_kernel(col2):
    mesh = plsc.VectorSubcoreMesh(core_axis_name="c", subcore_axis_name="s")

    @functools.partial(
        pl.kernel,
        out_type=jax.ShapeDtypeStruct((2 * NPAD,), _f32),
        mesh=mesh,
        compiler_params=pltpu.CompilerParams(use_tc_tiling_on_sc=False),
        scratch_types=[
            pltpu.VMEM_SHARED((NPAD,), _f32),
            pltpu.VMEM((CH,), _f32),
            pltpu.VMEM((TPN,), _f32),
            pltpu.VMEM((CH,), _i32),
        ],
    )
    def k(col_hbm, deg_hbm, cnt_sh, ones_v, stage_v, idx_v):
        cid = lax.axis_index("c")
        sid = lax.axis_index("s")
        base = sid * TPN

        @pl.loop(0, CH, step=16)
        def _(i):
            ones_v[pl.ds(i, 16)] = jnp.ones((16,), _f32)

        @pl.loop(0, TPN, step=16)
        def _(i):
            stage_v[pl.ds(i, 16)] = jnp.zeros((16,), _f32)

        pltpu.sync_copy(stage_v, cnt_sh.at[pl.ds(base, TPN)])
        plsc.subcore_barrier()

        @pl.loop(0, NCH2)
        def _(c):
            pltpu.sync_copy(col_hbm.at[cid, sid, c], idx_v)
            pltpu.sync_copy(ones_v, cnt_sh.at[idx_v], add=True)

        plsc.subcore_barrier()
        pltpu.sync_copy(cnt_sh.at[pl.ds(base, TPN)], stage_v)
        pltpu.sync_copy(stage_v, deg_hbm.at[pl.ds(cid * NPAD + base, TPN)])

    return k(col2)


def _prop_kernel(p0s, a2, sq2, row4, col4):
    mesh = plsc.VectorSubcoreMesh(core_axis_name="c", subcore_axis_name="s")

    @functools.partial(
        pl.kernel,
        out_type=jax.ShapeDtypeStruct((2 * NPAD, FH), _f32),
        mesh=mesh,
        compiler_params=pltpu.CompilerParams(use_tc_tiling_on_sc=False),
        scratch_types=[
            pltpu.VMEM_SHARED((NPAD, FH), _f32),
            pltpu.VMEM_SHARED((NPAD, FH), _f32),
            pltpu.VMEM((TPN, FH), _f32),
            pltpu.VMEM((TPN, FH), _f32),
            pltpu.VMEM((TPN, FH), _f32),
            pltpu.VMEM((CHB, FH), _f32),
            pltpu.VMEM((CHB, FH), _f32),
            pltpu.VMEM((CHB,), _i32),
            pltpu.VMEM((CHB,), _i32),
            pltpu.VMEM((CHB,), _i32),
            pltpu.VMEM((CHB,), _i32),
            pltpu.SemaphoreType.DMA,
            pltpu.SemaphoreType.DMA,
        ],
    )
    def k(p0_hbm, a2_hbm, sq2_hbm, row_hbm, col_hbm, out_hbm,
          t_sh, p_sh, p_v, p0_v, a2_v, gA, gB, riA, ciA, riB, ciB,
          semA, semB):
        cid = lax.axis_index("c")
        sid = lax.axis_index("s")
        base = sid * TPN
        gbase = cid * NPAD + base

        pltpu.sync_copy(p0_hbm.at[pl.ds(gbase, TPN)], p0_v)
        pltpu.sync_copy(a2_hbm.at[pl.ds(base, TPN)], a2_v)

        pltpu.sync_copy(p0_v, p_sh.at[pl.ds(base, TPN)])
        pltpu.sync_copy(p0_v, t_sh.at[pl.ds(base, TPN)])
        plsc.subcore_barrier()

        @pl.loop(0, K)
        def _(step):
            pltpu.sync_copy(row_hbm.at[sid, 0], riA)
            pltpu.sync_copy(col_hbm.at[sid, 0], ciA)
            pltpu.async_copy(p_sh.at[riA], gA, semA)

            @pl.loop(0, NCHB, step=2)
            def _(c):
                pltpu.sync_copy(row_hbm.at[sid, c + 1], riB)
                pltpu.sync_copy(col_hbm.at[sid, c + 1], ciB)
                pltpu.async_copy(p_sh.at[riB], gB, semB)
                pltpu.make_async_copy(p_sh.at[riA], gA, semA).wait()
                pltpu.sync_copy(gA, t_sh.at[ciA], add=True)

                @pl.when(c + 2 < NCHB)
                def _():
                    pltpu.sync_copy(row_hbm.at[sid, c + 2], riA)
                    pltpu.sync_copy(col_hbm.at[sid, c + 2], ciA)
                    pltpu.async_copy(p_sh.at[riA], gA, semA)

                pltpu.make_async_copy(p_sh.at[riB], gB, semB).wait()
                pltpu.sync_copy(gB, t_sh.at[ciB], add=True)

            plsc.subcore_barrier()

            pltpu.sync_copy(t_sh.at[pl.ds(base, CHB)], gA)
            pltpu.sync_copy(t_sh.at[pl.ds(base + CHB, TPN - CHB)],
                            gB.at[pl.ds(0, TPN - CHB)])

            @pl.loop(0, CHB)
            def _(r):
                p_v[r, pl.ds(0, 16)] = (gA[r, pl.ds(0, 16)]
                                        * a2_v[r, pl.ds(0, 16)]
                                        + p0_v[r, pl.ds(0, 16)] * ALPHA)
                p_v[r, pl.ds(16, 16)] = (gA[r, pl.ds(16, 16)]
                                         * a2_v[r, pl.ds(16, 16)]
                                         + p0_v[r, pl.ds(16, 16)] * ALPHA)

            @pl.loop(CHB, TPN)
            def _(r):
                p_v[r, pl.ds(0, 16)] = (gB[r - CHB, pl.ds(0, 16)]
                                        * a2_v[r, pl.ds(0, 16)]
                                        + p0_v[r, pl.ds(0, 16)] * ALPHA)
                p_v[r, pl.ds(16, 16)] = (gB[r - CHB, pl.ds(16, 16)]
                                         * a2_v[r, pl.ds(16, 16)]
                                         + p0_v[r, pl.ds(16, 16)] * ALPHA)

            pltpu.sync_copy(p_v, p_sh.at[pl.ds(base, TPN)])
            pltpu.sync_copy(p_v, t_sh.at[pl.ds(base, TPN)])
            plsc.subcore_barrier()

        pltpu.sync_copy(sq2_hbm.at[pl.ds(base, CHB)], gA)
        pltpu.sync_copy(sq2_hbm.at[pl.ds(base + CHB, TPN - CHB)],
                        gB.at[pl.ds(0, TPN - CHB)])

        @pl.loop(0, CHB)
        def _(r):
            p_v[r, pl.ds(0, 16)] = p_v[r, pl.ds(0, 16)] * gA[r, pl.ds(0, 16)]
            p_v[r, pl.ds(16, 16)] = (p_v[r, pl.ds(16, 16)]
                                     * gA[r, pl.ds(16, 16)])

        @pl.loop(CHB, TPN)
        def _(r):
            p_v[r, pl.ds(0, 16)] = (p_v[r, pl.ds(0, 16)]
                                    * gB[r - CHB, pl.ds(0, 16)])
            p_v[r, pl.ds(16, 16)] = (p_v[r, pl.ds(16, 16)]
                                     * gB[r - CHB, pl.ds(16, 16)])

        pltpu.sync_copy(p_v, out_hbm.at[pl.ds(gbase, TPN)])

    return k(p0s, a2, sq2, row4, col4)


def kernel(x, edge_index, W1, b1, W2, b2):
    row = edge_index[0].astype(_i32)
    col = edge_index[1].astype(_i32)
    pad = jnp.full((EPAD - E,), DUMMY, _i32)
    rowp = jnp.concatenate([row, pad])
    colp = jnp.concatenate([col, pad])

    col2 = colp.reshape(2, 16, NCH2, CH)
    col4 = colp.reshape(16, NCHB, CHB)
    row4 = rowp.reshape(16, NCHB, CHB)

    xp = jnp.pad(x, ((0, NPAD - N), (0, 0)))

    h = _mlp(xp, W1, b1, W2, b2)
    degp = _deg_kernel(col2)
    p0s, a2, sq2 = _prep(h, degp.reshape(2, NPAD))
    out2 = _prop_kernel(
        p0s.reshape(2 * NPAD, FH), a2, sq2, row4, col4)

    out2 = out2.reshape(2, NPAD, FH)
    return jnp.concatenate([out2[0, :N, :], out2[1, :N, :]], axis=1)


# speedup vs baseline: 27.2262x; 1.3585x over previous
import functools

import jax
import jax.numpy as jnp
from jax import lax
from jax.experimental import pallas as pl
from jax.experimental.pallas import tpu as pltpu
from jax.experimental.pallas import tpu_sc as plsc

N = 10000
E = 320000
D_IN = 128
D_HID = 128
D_OUT = 64
K = 10
ALPHA = 0.1

NPAD = 10240
TPN = NPAD // 16
FH = 32
DUMMY = N

EPT = 20480
EPAD = 16 * EPT
CH = 1024
CHB = 320
NCHB = EPT // CHB

EPT2 = EPAD // 32
NCH2 = EPT2 // CH

_f32 = jnp.float32
_i32 = jnp.int32


def _mlp_body(x_ref, w1_ref, b1_ref, w2_ref, b2_ref, h_ref):
    hmid = jnp.dot(x_ref[...], w1_ref[...],
                   preferred_element_type=jnp.float32,
                   precision=lax.Precision.HIGHEST)
    hmid = jnp.maximum(hmid + b1_ref[...], 0.0)
    h = jnp.dot(hmid, w2_ref[...],
                preferred_element_type=jnp.float32,
                precision=lax.Precision.HIGHEST)
    h_ref[...] = h + b2_ref[...]


def _mlp(xp, W1, b1, W2, b2):
    return pl.pallas_call(
        _mlp_body,
        out_shape=jax.ShapeDtypeStruct((NPAD, D_OUT), _f32),
    )(xp, W1, b1.reshape(1, D_HID), W2, b2.reshape(1, D_OUT))


def _prep_body(h_ref, degp_ref, p0_ref, a2_ref, sq2_ref):
    deg = degp_ref[0, :] + degp_ref[1, :] + 1.0
    dinv = lax.rsqrt(deg)
    a2_ref[...] = jnp.broadcast_to(
        ((1.0 - ALPHA) * dinv * dinv)[:, None], (NPAD, FH))
    sq2_ref[...] = jnp.broadcast_to((deg * dinv)[:, None], (NPAD, FH))
    p0 = h_ref[...] * dinv[:, None]
    p0_ref[0] = p0[:, :FH]
    p0_ref[1] = p0[:, FH:]


def _prep(h, degp):
    return pl.pallas_call(
        _prep_body,
        out_shape=[
            jax.ShapeDtypeStruct((2, NPAD, FH), _f32),
            jax.ShapeDtypeStruct((NPAD, FH), _f32),
            jax.ShapeDtypeStruct((NPAD, FH), _f32),
        ],
    )(h, degp)


def _deg_kernel(col2):
    mesh = plsc.VectorSubcoreMesh(core_axis_name="c", subcore_axis_name="s")

    @functools.partial(
        pl.kernel,
        out_type=jax.ShapeDtypeStruct((2 * NPAD,), _f32),
        mesh=mesh,
        compiler_params=pltpu.CompilerParams(use_tc_tiling_on_sc=False),
        scratch_types=[
            pltpu.VMEM_SHARED((NPAD,), _f32),
            pltpu.VMEM((CH,), _f32),
            pltpu.VMEM((TPN,), _f32),
            pltpu.VMEM((CH,), _i32),
        ],
    )
    def k(col_hbm, deg_hbm, cnt_sh, ones_v, stage_v, idx_v):
        cid = lax.axis_index("c")
        sid = lax.axis_index("s")
        base = sid * TPN

        @pl.loop(0, CH, step=16)
        def _(i):
            ones_v[pl.ds(i, 16)] = jnp.ones((16,), _f32)

        @pl.loop(0, TPN, step=16)
        def _(i):
            stage_v[pl.ds(i, 16)] = jnp.zeros((16,), _f32)

        pltpu.sync_copy(stage_v, cnt_sh.at[pl.ds(base, TPN)])
        plsc.subcore_barrier()

        @pl.loop(0, NCH2)
        def _(c):
            pltpu.sync_copy(col_hbm.at[cid, sid, c], idx_v)
            pltpu.sync_copy(ones_v, cnt_sh.at[idx_v], add=True)

        plsc.subcore_barrier()
        pltpu.sync_copy(cnt_sh.at[pl.ds(base, TPN)], stage_v)
        pltpu.sync_copy(stage_v, deg_hbm.at[pl.ds(cid * NPAD + base, TPN)])

    return k(col2)


def _prop_kernel(p0s, a2, sq2, row4, col4):
    mesh = plsc.VectorSubcoreMesh(core_axis_name="c", subcore_axis_name="s")

    @functools.partial(
        pl.kernel,
        out_type=jax.ShapeDtypeStruct((2 * NPAD, FH), _f32),
        mesh=mesh,
        compiler_params=pltpu.CompilerParams(use_tc_tiling_on_sc=False),
        scratch_types=[
            pltpu.VMEM_SHARED((NPAD, FH), _f32),
            pltpu.VMEM_SHARED((NPAD, FH), _f32),
            pltpu.VMEM((TPN, FH), _f32),
            pltpu.VMEM((TPN, FH), _f32),
            pltpu.VMEM((TPN, FH), _f32),
            pltpu.VMEM((CHB, FH), _f32),
            pltpu.VMEM((CHB, FH), _f32),
            pltpu.VMEM((CHB,), _i32),
            pltpu.VMEM((CHB,), _i32),
            pltpu.VMEM((CHB,), _i32),
            pltpu.VMEM((CHB,), _i32),
            pltpu.SemaphoreType.DMA,
            pltpu.SemaphoreType.DMA,
            pltpu.SemaphoreType.DMA,
            pltpu.SemaphoreType.DMA,
        ],
    )
    def k(p0_hbm, a2_hbm, sq2_hbm, row_hbm, col_hbm, out_hbm,
          t_sh, p_sh, p_v, p0_v, a2_v, gA, gB, riA, ciA, riB, ciB,
          semA, semB, semIA, semIB):
        cid = lax.axis_index("c")
        sid = lax.axis_index("s")
        base = sid * TPN
        gbase = cid * NPAD + base

        pltpu.sync_copy(p0_hbm.at[pl.ds(gbase, TPN)], p0_v)
        pltpu.sync_copy(a2_hbm.at[pl.ds(base, TPN)], a2_v)

        pltpu.sync_copy(p0_v, p_sh.at[pl.ds(base, TPN)])
        pltpu.sync_copy(p0_v, t_sh.at[pl.ds(base, TPN)])
        plsc.subcore_barrier()

        @pl.loop(0, K)
        def _(step):
            pltpu.sync_copy(row_hbm.at[sid, 0], riA)
            pltpu.sync_copy(col_hbm.at[sid, 0], ciA)
            pltpu.async_copy(p_sh.at[riA], gA, semA)
            pltpu.async_copy(row_hbm.at[sid, 1], riB, semIB)
            pltpu.async_copy(col_hbm.at[sid, 1], ciB, semIB)

            @pl.loop(0, NCHB, step=2)
            def _(c):
                pltpu.make_async_copy(row_hbm.at[sid, c + 1], riB, semIB).wait()
                pltpu.make_async_copy(col_hbm.at[sid, c + 1], ciB, semIB).wait()
                pltpu.async_copy(p_sh.at[riB], gB, semB)
                pltpu.make_async_copy(p_sh.at[riA], gA, semA).wait()
                pltpu.sync_copy(gA, t_sh.at[ciA], add=True)

                @pl.when(c + 2 < NCHB)
                def _():
                    pltpu.async_copy(row_hbm.at[sid, c + 2], riA, semIA)
                    pltpu.async_copy(col_hbm.at[sid, c + 2], ciA, semIA)

                pltpu.make_async_copy(p_sh.at[riB], gB, semB).wait()
                pltpu.sync_copy(gB, t_sh.at[ciB], add=True)

                @pl.when(c + 2 < NCHB)
                def _():
                    pltpu.make_async_copy(row_hbm.at[sid, c + 2], riA,
                                          semIA).wait()
                    pltpu.make_async_copy(col_hbm.at[sid, c + 2], ciA,
                                          semIA).wait()
                    pltpu.async_copy(p_sh.at[riA], gA, semA)
                    pltpu.async_copy(row_hbm.at[sid, c + 3], riB, semIB)
                    pltpu.async_copy(col_hbm.at[sid, c + 3], ciB, semIB)

            plsc.subcore_barrier()

            pltpu.sync_copy(t_sh.at[pl.ds(base, CHB)], gA)
            pltpu.sync_copy(t_sh.at[pl.ds(base + CHB, TPN - CHB)],
                            gB.at[pl.ds(0, TPN - CHB)])

            @pl.loop(0, CHB)
            def _(r):
                p_v[r, pl.ds(0, 16)] = (gA[r, pl.ds(0, 16)]
                                        * a2_v[r, pl.ds(0, 16)]
                                        + p0_v[r, pl.ds(0, 16)] * ALPHA)
                p_v[r, pl.ds(16, 16)] = (gA[r, pl.ds(16, 16)]
                                         * a2_v[r, pl.ds(16, 16)]
                                         + p0_v[r, pl.ds(16, 16)] * ALPHA)

            @pl.loop(CHB, TPN)
            def _(r):
                p_v[r, pl.ds(0, 16)] = (gB[r - CHB, pl.ds(0, 16)]
                                        * a2_v[r, pl.ds(0, 16)]
                                        + p0_v[r, pl.ds(0, 16)] * ALPHA)
                p_v[r, pl.ds(16, 16)] = (gB[r - CHB, pl.ds(16, 16)]
                                         * a2_v[r, pl.ds(16, 16)]
                                         + p0_v[r, pl.ds(16, 16)] * ALPHA)

            pltpu.sync_copy(p_v, p_sh.at[pl.ds(base, TPN)])
            pltpu.sync_copy(p_v, t_sh.at[pl.ds(base, TPN)])
            plsc.subcore_barrier()

        pltpu.sync_copy(sq2_hbm.at[pl.ds(base, CHB)], gA)
        pltpu.sync_copy(sq2_hbm.at[pl.ds(base + CHB, TPN - CHB)],
                        gB.at[pl.ds(0, TPN - CHB)])

        @pl.loop(0, CHB)
        def _(r):
            p_v[r, pl.ds(0, 16)] = p_v[r, pl.ds(0, 16)] * gA[r, pl.ds(0, 16)]
            p_v[r, pl.ds(16, 16)] = (p_v[r, pl.ds(16, 16)]
                                     * gA[r, pl.ds(16, 16)])

        @pl.loop(CHB, TPN)
        def _(r):
            p_v[r, pl.ds(0, 16)] = (p_v[r, pl.ds(0, 16)]
                                    * gB[r - CHB, pl.ds(0, 16)])
            p_v[r, pl.ds(16, 16)] = (p_v[r, pl.ds(16, 16)]
                                     * gB[r - CHB, pl.ds(16, 16)])

        pltpu.sync_copy(p_v, out_hbm.at[pl.ds(gbase, TPN)])

    return k(p0s, a2, sq2, row4, col4)


def kernel(x, edge_index, W1, b1, W2, b2):
    row = edge_index[0].astype(_i32)
    col = edge_index[1].astype(_i32)
    pad = jnp.full((EPAD - E,), DUMMY, _i32)
    rowp = jnp.concatenate([row, pad])
    colp = jnp.concatenate([col, pad])

    col2 = colp.reshape(2, 16, NCH2, CH)
    col4 = colp.reshape(16, NCHB, CHB)
    row4 = rowp.reshape(16, NCHB, CHB)

    xp = jnp.pad(x, ((0, NPAD - N), (0, 0)))

    h = _mlp(xp, W1, b1, W2, b2)
    degp = _deg_kernel(col2)
    p0s, a2, sq2 = _prep(h, degp.reshape(2, NPAD))
    out2 = _prop_kernel(
        p0s.reshape(2 * NPAD, FH), a2, sq2, row4, col4)

    out2 = out2.reshape(2, NPAD, FH)
    return jnp.concatenate([out2[0, :N, :], out2[1, :N, :]], axis=1)
